# baseline (speedup 1.0000x reference)
import jax
import jax.numpy as jnp
from jax import lax
from jax.experimental import pallas as pl
from jax.experimental.pallas import tpu as pltpu
from jax.experimental.pallas import tpu_sc as plsc

N = 10000
E = 320000
F_IN = 128
HID = 64
HEADS = 8
C = 40

NP = 10240
PADN = N
NC, NS, L = 2, 16, 16
NW = NC * NS
ROWS_PT = NP // NS
EP = 344064
EPT = EP // NW
EPT2 = EP // NS
BLK = 128
NBLK = EPT // BLK
NBLK2 = EPT2 // BLK
BN = 2560
F32 = jnp.float32

_SC_PARAMS = pltpu.CompilerParams(
    use_tc_tiling_on_sc=False, needs_layout_passes=False)


def _t1_body(x_ref, w1_ref, wsd_ref, h4_ref, lg_ref):
    k = pl.program_id(1)
    hblk = jnp.dot(x_ref[...], w1_ref[...], preferred_element_type=F32)
    h4_ref[...] = hblk[None]
    part = jnp.dot(hblk, wsd_ref[...], preferred_element_type=F32)

    @pl.when(k == 0)
    def _():
        lg_ref[...] = part

    @pl.when(k > 0)
    def _():
        lg_ref[...] += part


def _t1(xp, W1, Wsd):
    grid = (NP // BN, 4)
    return pl.pallas_call(
        _t1_body,
        grid=grid,
        in_specs=[
            pl.BlockSpec((BN, 128), lambda i, k: (i, 0)),
            pl.BlockSpec((128, 128), lambda i, k: (0, k)),
            pl.BlockSpec((128, 32), lambda i, k: (k, 0)),
        ],
        out_specs=[
            pl.BlockSpec((1, BN, 128), lambda i, k: (k, i, 0)),
            pl.BlockSpec((BN, 32), lambda i, k: (i, 0)),
        ],
        out_shape=[
            jax.ShapeDtypeStruct((4, NP, 128), F32),
            jax.ShapeDtypeStruct((NP, 32), F32),
        ],
    )(xp, W1, Wsd)


def _t2_body(o1_ref, dr_ref, r_ref, b1_ref, w2_ref, a2_ref, h2_ref, lg2_ref):
    k = pl.program_id(1)
    rk = jax.lax.broadcasted_iota(jnp.int32, (8, 1), 0) == k
    bsel = jnp.sum(jnp.where(rk, b1_ref[...], 0.0), axis=0, keepdims=True)
    drv = 1.0 / (dr_ref[0] + dr_ref[1] + 1e-16)
    scale = jnp.dot(drv, r_ref[...], preferred_element_type=F32)
    v = o1_ref[0] * scale + bsel
    ek = jnp.where(v > 0, v, jnp.exp(v) - 1.0)
    ph = jnp.dot(ek, w2_ref[...], preferred_element_type=F32)
    wa = jnp.dot(w2_ref[...], a2_ref[...], preferred_element_type=F32)
    pl2 = jnp.dot(ek, wa, preferred_element_type=F32)

    @pl.when(k == 0)
    def _():
        h2_ref[...] = ph
        lg2_ref[...] = pl2

    @pl.when(k > 0)
    def _():
        h2_ref[...] += ph
        lg2_ref[...] += pl2


def _t2(out1, drec1, R1, b1r, W2p, A2):
    grid = (NP // BN, 4)
    return pl.pallas_call(
        _t2_body,
        grid=grid,
        in_specs=[
            pl.BlockSpec((1, BN, 128), lambda i, k: (k, i, 0)),
            pl.BlockSpec((2, BN, 16), lambda i, k: (0, i, 0)),
            pl.BlockSpec((16, 128), lambda i, k: (0, k)),
            pl.BlockSpec((8, 128), lambda i, k: (0, 0)),
            pl.BlockSpec((128, 48), lambda i, k: (k, 0)),
            pl.BlockSpec((48, 32), lambda i, k: (0, 0)),
        ],
        out_specs=[
            pl.BlockSpec((BN, 48), lambda i, k: (i, 0)),
            pl.BlockSpec((BN, 32), lambda i, k: (i, 0)),
        ],
        out_shape=[
            jax.ShapeDtypeStruct((NP, 48), F32),
            jax.ShapeDtypeStruct((NP, 32), F32),
        ],
    )(out1, drec1, R1, b1r, W2p, A2)


def _trecip_body(p_ref, o_ref):
    o_ref[...] = 1.0 / (p_ref[0] + p_ref[1] + 1e-16)


def _trecip(parts):
    return pl.pallas_call(
        _trecip_body,
        grid=(NP // BN,),
        in_specs=[pl.BlockSpec((2, BN, 16), lambda i: (0, i, 0))],
        out_specs=pl.BlockSpec((BN, 16), lambda i: (i, 0)),
        out_shape=jax.ShapeDtypeStruct((NP, 16), F32),
    )(parts)


def _t3_body(p_ref, dr_ref, o_ref, b2_ref, out_ref):
    drv = 1.0 / (dr_ref[0] + dr_ref[1] + 1e-16)
    scale = jnp.dot(drv, o_ref[...], preferred_element_type=F32)
    s = (p_ref[0] + p_ref[1]) * scale + b2_ref[...]
    out_ref[...] = s[:, :40]


def _t3(p2, drec2, O2, b2p):
    BT = 2000
    return pl.pallas_call(
        _t3_body,
        grid=(N // BT,),
        in_specs=[
            pl.BlockSpec((2, BT, 48), lambda i: (0, i, 0)),
            pl.BlockSpec((2, BT, 16), lambda i: (0, i, 0)),
            pl.BlockSpec((16, 48), lambda i: (0, 0)),
            pl.BlockSpec((1, 48), lambda i: (0, 0)),
        ],
        out_specs=pl.BlockSpec((BT, 40), lambda i: (i, 0)),
        out_shape=jax.ShapeDtypeStruct((N, 40), F32),
    )(p2, drec2, O2, b2p)


_MESH = plsc.VectorSubcoreMesh(
    core_axis_name="c", subcore_axis_name="s", num_cores=NC, num_subcores=NS)


def _sca_body(srcA_h, dstA_h, ts_h, td_h, e_h, dp_h,
              idxs_v, idxd_v, gs_v, gd_v, ev_v, zb_v, acc_sh,
              sg0, sg1, sd0, sd1):
    c = lax.axis_index("c")
    s = lax.axis_index("s")
    w = s * NC + c
    sgs = (sg0, sg1)
    sds = (sd0, sd1)

    def zrow(i, _):
        zb_v[i, :] = jnp.zeros((L,), F32)
        return 0
    lax.fori_loop(0, ROWS_PT, zrow, 0)
    pltpu.sync_copy(zb_v, acc_sh.at[pl.ds(s * ROWS_PT, ROWS_PT)])
    pltpu.sync_copy(srcA_h.at[w], idxs_v)
    pltpu.sync_copy(dstA_h.at[w], idxd_v)
    plsc.subcore_barrier()

    for q in range(2):
        pltpu.async_copy(ts_h.at[idxs_v.at[q]], gs_v.at[q], sgs[q])
        pltpu.async_copy(td_h.at[idxd_v.at[q]], gd_v.at[q], sds[q])

    def group(g, _):
        for q in range(2):
            b = 2 * g + q
            pltpu.make_async_copy(
                ts_h.at[idxs_v.at[0]], gs_v.at[q], sgs[q]).wait()
            pltpu.make_async_copy(
                td_h.at[idxd_v.at[0]], gd_v.at[q], sds[q]).wait()

            def erow(i, _):
                a = gs_v[q, i, :] + gd_v[q, i, :]
                a = jnp.where(a > 0, a, 0.2 * a)
                ev_v[i, :] = jnp.exp(a)
                return 0
            lax.fori_loop(0, BLK, erow, 0)
            pltpu.sync_copy(ev_v, e_h.at[pl.ds(w * EPT + b * BLK, BLK)])
            pltpu.sync_copy(ev_v, acc_sh.at[idxd_v.at[b]], add=True)
            nb = b + 2

            @pl.when(nb < NBLK)
            def _():
                pltpu.async_copy(ts_h.at[idxs_v.at[nb]], gs_v.at[q], sgs[q])
                pltpu.async_copy(td_h.at[idxd_v.at[nb]], gd_v.at[q], sds[q])
        return 0
    lax.fori_loop(0, NBLK // 2, group, 0)
    plsc.subcore_barrier()
    pltpu.sync_copy(acc_sh.at[pl.ds(s * ROWS_PT, ROWS_PT)],
                    dp_h.at[c, pl.ds(s * ROWS_PT, ROWS_PT)])


_sca = pl.kernel(
    _sca_body,
    out_type=[
        jax.ShapeDtypeStruct((EP, 16), F32),
        jax.ShapeDtypeStruct((2, NP, 16), F32),
    ],
    mesh=_MESH,
    compiler_params=_SC_PARAMS,
    scratch_types=[
        pltpu.VMEM((NBLK, BLK), jnp.int32),
        pltpu.VMEM((NBLK, BLK), jnp.int32),
        pltpu.VMEM((2, BLK, 16), F32),
        pltpu.VMEM((2, BLK, 16), F32),
        pltpu.VMEM((BLK, 16), F32),
        pltpu.VMEM((ROWS_PT, 16), F32),
        pltpu.VMEM_SHARED((NP, 16), F32),
        pltpu.SemaphoreType.DMA,
        pltpu.SemaphoreType.DMA,
        pltpu.SemaphoreType.DMA,
        pltpu.SemaphoreType.DMA,
    ],
)


def _scb_body(srcB_h, dstB_h, e_h, h4_h, out_h,
              idxs_v, idxd_v, ev_v, rows_v,
              si0, si1, si2, si3, sr0, sr1, se0, se1, acc_sh):
    c = lax.axis_index("c")
    s = lax.axis_index("s")
    sis = (si0, si1, si2, si3)
    srs = (sr0, sr1)
    ses = (se0, se1)

    for p in range(2):
        def zrow(i, _):
            for j in range(8):
                rows_v[0, i, pl.ds(16 * j, 16)] = jnp.zeros((L,), F32)
            return 0
        lax.fori_loop(0, BLK, zrow, 0)
        for q in range(5):
            pltpu.sync_copy(rows_v.at[0],
                            acc_sh.at[pl.ds(s * ROWS_PT + q * BLK, BLK)])
        plsc.subcore_barrier()

        for cc in range(2):
            @pl.when(c == cc)
            def _(p=p, cc=cc):
                k = 2 * cc + p
                for j in range(2):
                    pltpu.sync_copy(srcB_h.at[s, j], idxs_v.at[j])
                    pltpu.sync_copy(dstB_h.at[s, j], idxd_v.at[j])
                for j in range(2, 4):
                    pltpu.async_copy(srcB_h.at[s, j], idxs_v.at[j], sis[j])
                    pltpu.async_copy(dstB_h.at[s, j], idxd_v.at[j], sis[j])
                for q in range(2):
                    pltpu.async_copy(h4_h.at[k].at[idxs_v.at[q]],
                                     rows_v.at[q], srs[q])
                    pltpu.async_copy(e_h.at[pl.ds(s * EPT2 + q * BLK, BLK)],
                                     ev_v.at[q], ses[q])

                def group(g, _):
                    for q in range(4):
                        b = 4 * g + q
                        q2 = q % 2
                        pltpu.make_async_copy(
                            h4_h.at[k].at[idxs_v.at[0]], rows_v.at[q2],
                            srs[q2]).wait()
                        pltpu.make_async_copy(
                            e_h.at[pl.ds(0, BLK)], ev_v.at[q2],
                            ses[q2]).wait()

                        def edge(i, _):
                            av = ev_v[q2, i, :]
                            b0 = jnp.broadcast_to(av[2 * k], (L,))
                            b1 = jnp.broadcast_to(av[2 * k + 1], (L,))
                            for j in range(4):
                                rows_v[q2, i, pl.ds(16 * j, 16)] = (
                                    rows_v[q2, i, pl.ds(16 * j, 16)] * b0)
                            for j in range(4, 8):
                                rows_v[q2, i, pl.ds(16 * j, 16)] = (
                                    rows_v[q2, i, pl.ds(16 * j, 16)] * b1)
                            return 0
                        lax.fori_loop(0, BLK, edge, 0, unroll=2)
                        pltpu.sync_copy(rows_v.at[q2],
                                        acc_sh.at[idxd_v.at[q]], add=True)

                        @pl.when(b + 4 < NBLK2)
                        def _():
                            pltpu.async_copy(srcB_h.at[s, b + 4],
                                             idxs_v.at[q], sis[q])
                            pltpu.async_copy(dstB_h.at[s, b + 4],
                                             idxd_v.at[q], sis[q])

                        @pl.when(b + 2 < NBLK2)
                        def _():
                            qn = (q + 2) % 4
                            pltpu.make_async_copy(
                                srcB_h.at[s, 0], idxs_v.at[qn], sis[qn]).wait()
                            pltpu.make_async_copy(
                                dstB_h.at[s, 0], idxd_v.at[qn], sis[qn]).wait()
                            pltpu.async_copy(
                                h4_h.at[k].at[idxs_v.at[qn]], rows_v.at[q2],
                                srs[q2])
                            pltpu.async_copy(
                                e_h.at[pl.ds(s * EPT2 + (b + 2) * BLK, BLK)],
                                ev_v.at[q2], ses[q2])
                    return 0
                lax.fori_loop(0, NBLK2 // 4, group, 0)
        plsc.subcore_barrier()
        k_dyn = 2 * c + p
        pltpu.sync_copy(acc_sh.at[pl.ds(s * ROWS_PT, ROWS_PT)],
                        out_h.at[k_dyn, pl.ds(s * ROWS_PT, ROWS_PT)])
        plsc.subcore_barrier()


_scb = pl.kernel(
    _scb_body,
    out_type=[jax.ShapeDtypeStruct((4, NP, 128), F32)],
    mesh=_MESH,
    compiler_params=_SC_PARAMS,
    scratch_types=[
        pltpu.VMEM((4, BLK), jnp.int32),
        pltpu.VMEM((4, BLK), jnp.int32),
        pltpu.VMEM((2, BLK, 16), F32),
        pltpu.VMEM((2, BLK, 128), F32),
        pltpu.SemaphoreType.DMA,
        pltpu.SemaphoreType.DMA,
        pltpu.SemaphoreType.DMA,
        pltpu.SemaphoreType.DMA,
        pltpu.SemaphoreType.DMA,
        pltpu.SemaphoreType.DMA,
        pltpu.SemaphoreType.DMA,
        pltpu.SemaphoreType.DMA,
        pltpu.VMEM_SHARED((NP, 128), F32),
    ],
)


def _scc2_body(srcA_h, dstA_h, e_h, h2_h, p2_h,
               idxs_v, idxd_v, ev_v, rows_v,
               sr0, sr1, se0, se1, acc_sh):
    c = lax.axis_index("c")
    s = lax.axis_index("s")
    w = s * NC + c
    srs = (sr0, sr1)
    ses = (se0, se1)

    pltpu.sync_copy(srcA_h.at[w], idxs_v)
    pltpu.sync_copy(dstA_h.at[w], idxd_v)

    def zrow(i, _):
        for j in range(3):
            rows_v[0, i, pl.ds(16 * j, 16)] = jnp.zeros((L,), F32)
        return 0
    lax.fori_loop(0, BLK, zrow, 0)
    for q in range(5):
        pltpu.sync_copy(rows_v.at[0],
                        acc_sh.at[pl.ds(s * ROWS_PT + q * BLK, BLK)])
    plsc.subcore_barrier()

    for q in range(2):
        pltpu.async_copy(h2_h.at[idxs_v.at[q]], rows_v.at[q], srs[q])
        pltpu.async_copy(e_h.at[pl.ds(w * EPT + q * BLK, BLK)],
                         ev_v.at[q], ses[q])

    def group(g, _):
        for q in range(2):
            b = 2 * g + q
            pltpu.make_async_copy(
                h2_h.at[idxs_v.at[0]], rows_v.at[q], srs[q]).wait()
            pltpu.make_async_copy(
                e_h.at[pl.ds(0, BLK)], ev_v.at[q], ses[q]).wait()

            def edge(i, _):
                av = ev_v[q, i, :]
                b0 = jnp.broadcast_to(av[0], (L,))
                for j in range(3):
                    rows_v[q, i, pl.ds(16 * j, 16)] = (
                        rows_v[q, i, pl.ds(16 * j, 16)] * b0)
                return 0
            lax.fori_loop(0, BLK, edge, 0, unroll=2)
            pltpu.sync_copy(rows_v.at[q], acc_sh.at[idxd_v.at[b]], add=True)
            nb = b + 2

            @pl.when(nb < NBLK)
            def _():
                pltpu.async_copy(h2_h.at[idxs_v.at[nb]], rows_v.at[q], srs[q])
                pltpu.async_copy(e_h.at[pl.ds(w * EPT + nb * BLK, BLK)],
                                 ev_v.at[q], ses[q])
        return 0
    lax.fori_loop(0, NBLK // 2, group, 0)
    plsc.subcore_barrier()
    pltpu.sync_copy(acc_sh.at[pl.ds(s * ROWS_PT, ROWS_PT)],
                    p2_h.at[c, pl.ds(s * ROWS_PT, ROWS_PT)])


_scc2 = pl.kernel(
    _scc2_body,
    out_type=[jax.ShapeDtypeStruct((2, NP, 48), F32)],
    mesh=_MESH,
    compiler_params=_SC_PARAMS,
    scratch_types=[
        pltpu.VMEM((NBLK, BLK), jnp.int32),
        pltpu.VMEM((NBLK, BLK), jnp.int32),
        pltpu.VMEM((2, BLK, 16), F32),
        pltpu.VMEM((2, BLK, 48), F32),
        pltpu.SemaphoreType.DMA,
        pltpu.SemaphoreType.DMA,
        pltpu.SemaphoreType.DMA,
        pltpu.SemaphoreType.DMA,
        pltpu.VMEM_SHARED((NP, 48), F32),
    ],
)


@jax.jit
def kernel(x, edge_index, W1, a_src1, a_dst1, b1, W2, a_src2, a_dst2, b2):
    loop = jnp.arange(N, dtype=edge_index.dtype)
    padfill = jnp.full((EP - E - N,), PADN, edge_index.dtype)
    srcp = jnp.concatenate([edge_index[0], loop, padfill])
    dstp = jnp.concatenate([edge_index[1], loop, padfill])
    srcA = srcp.reshape(NW, NBLK, BLK)
    dstA = dstp.reshape(NW, NBLK, BLK)
    srcB = srcp.reshape(NS, NBLK2, BLK)
    dstB = dstp.reshape(NS, NBLK2, BLK)
    xp = jnp.zeros((NP, F_IN), F32).at[:N].set(x)

    eye = jnp.eye(HEADS, dtype=F32)
    S_src = (a_src1[:, :, None] * eye[:, None, :]).reshape(512, 8)
    S_dst = (a_dst1[:, :, None] * eye[:, None, :]).reshape(512, 8)
    zc = jnp.zeros((512, 8), F32)
    Wsd = jnp.concatenate([S_src, zc, S_dst, zc], axis=1)

    b1r = jnp.pad(b1.reshape(4, 128), ((0, 4), (0, 0)))
    W2p = jnp.pad(W2, ((0, 0), (0, 8)))
    a2s = jnp.tile(jnp.pad(a_src2[0], (0, 8))[:, None], (1, 16))
    a2d = jnp.tile(jnp.pad(a_dst2[0], (0, 8))[:, None], (1, 16))
    A2 = jnp.concatenate([a2s, a2d], axis=1)
    b2p = jnp.pad(b2, (0, 8)).reshape(1, 48)

    R1m = jnp.pad(jnp.repeat(eye, 64, axis=1), ((0, 8), (0, 0)))
    O2m = jnp.zeros((16, 48), F32).at[0, :].set(1.0)

    h4, lg = _t1(xp, W1, Wsd)
    a1s, a1d = lg[:, :16], lg[:, 16:]
    e1, d1p = _sca(srcA, dstA, a1s, a1d)
    (out1,) = _scb(srcB, dstB, e1, h4)

    h2p, lg2 = _t2(out1, d1p, R1m, b1r, W2p, A2)
    a2s_t, a2d_t = lg2[:, :16], lg2[:, 16:]
    e2, d2p = _sca(srcA, dstA, a2s_t, a2d_t)
    (p2,) = _scc2(srcA, dstA, e2, h2p)

    return _t3(p2, d2p, O2m, b2p)


# speedup vs baseline: 2.2279x; 2.2279x over previous
import jax
import jax.numpy as jnp
from jax import lax
from jax.experimental import pallas as pl
from jax.experimental.pallas import tpu as pltpu
from jax.experimental.pallas import tpu_sc as plsc

N = 10000
E = 320000
F_IN = 128
HID = 64
HEADS = 8
C = 40

NP = 10240
PADN = N
NC, NS, L = 2, 16, 16
NW = NC * NS
ROWS_PT = NP // NS
EP = 344064
EPT = EP // NW
EPT2 = EP // NS
BLK = 128
NBLK = EPT // BLK
BLKB = 112
NBB = EPT2 // BLKB
NPA = NP
RPT_A = NPA // NS
BN = 2560
F32 = jnp.float32

_SC_PARAMS = pltpu.CompilerParams(
    use_tc_tiling_on_sc=False, needs_layout_passes=False)


def _t1_body(x_ref, w1_ref, wsd_ref, h4_ref, lg_ref):
    k = pl.program_id(1)
    hblk = jnp.dot(x_ref[...], w1_ref[...], preferred_element_type=F32)
    h4_ref[...] = hblk[None]
    part = jnp.dot(hblk, wsd_ref[...], preferred_element_type=F32)

    @pl.when(k == 0)
    def _():
        lg_ref[...] = part

    @pl.when(k > 0)
    def _():
        lg_ref[...] += part


def _t1(xp, W1, Wsd):
    grid = (NP // BN, 4)
    return pl.pallas_call(
        _t1_body,
        grid=grid,
        in_specs=[
            pl.BlockSpec((BN, 128), lambda i, k: (i, 0)),
            pl.BlockSpec((128, 128), lambda i, k: (0, k)),
            pl.BlockSpec((128, 32), lambda i, k: (k, 0)),
        ],
        out_specs=[
            pl.BlockSpec((1, BN, 128), lambda i, k: (k, i, 0)),
            pl.BlockSpec((BN, 32), lambda i, k: (i, 0)),
        ],
        out_shape=[
            jax.ShapeDtypeStruct((4, NP, 128), F32),
            jax.ShapeDtypeStruct((NP, 32), F32),
        ],
    )(xp, W1, Wsd)


def _t2_body(o1_ref, dr_ref, r_ref, b1_ref, w2_ref, a2_ref, h2_ref, lg2_ref):
    k = pl.program_id(1)
    rk = jax.lax.broadcasted_iota(jnp.int32, (8, 1), 0) == k
    bsel = jnp.sum(jnp.where(rk, b1_ref[...], 0.0), axis=0, keepdims=True)
    drv = 1.0 / (dr_ref[0] + dr_ref[1] + 1e-16)
    scale = jnp.dot(drv, r_ref[...], preferred_element_type=F32)
    v = o1_ref[0] * scale + bsel
    ek = jnp.where(v > 0, v, jnp.exp(v) - 1.0)
    ph = jnp.dot(ek, w2_ref[...], preferred_element_type=F32)
    wa = jnp.dot(w2_ref[...], a2_ref[...], preferred_element_type=F32)
    pl2 = jnp.dot(ek, wa, preferred_element_type=F32)

    @pl.when(k == 0)
    def _():
        h2_ref[...] = ph
        lg2_ref[...] = pl2

    @pl.when(k > 0)
    def _():
        h2_ref[...] += ph
        lg2_ref[...] += pl2


def _t2(out1, drec1, R1, b1r, W2p, A2):
    grid = (NP // BN, 4)
    return pl.pallas_call(
        _t2_body,
        grid=grid,
        in_specs=[
            pl.BlockSpec((1, BN, 128), lambda i, k: (k, i, 0)),
            pl.BlockSpec((2, BN, 16), lambda i, k: (0, i, 0)),
            pl.BlockSpec((16, 128), lambda i, k: (0, k)),
            pl.BlockSpec((8, 128), lambda i, k: (0, 0)),
            pl.BlockSpec((128, 48), lambda i, k: (k, 0)),
            pl.BlockSpec((48, 32), lambda i, k: (0, 0)),
        ],
        out_specs=[
            pl.BlockSpec((BN, 48), lambda i, k: (i, 0)),
            pl.BlockSpec((BN, 32), lambda i, k: (i, 0)),
        ],
        out_shape=[
            jax.ShapeDtypeStruct((NP, 48), F32),
            jax.ShapeDtypeStruct((NP, 32), F32),
        ],
    )(out1, drec1, R1, b1r, W2p, A2)


def _trecip_body(p_ref, o_ref):
    o_ref[...] = 1.0 / (p_ref[0] + p_ref[1] + 1e-16)


def _trecip(parts):
    return pl.pallas_call(
        _trecip_body,
        grid=(NP // BN,),
        in_specs=[pl.BlockSpec((2, BN, 16), lambda i: (0, i, 0))],
        out_specs=pl.BlockSpec((BN, 16), lambda i: (i, 0)),
        out_shape=jax.ShapeDtypeStruct((NP, 16), F32),
    )(parts)


def _t3_body(p_ref, dr_ref, o_ref, b2_ref, out_ref):
    drv = 1.0 / (dr_ref[0] + dr_ref[1] + 1e-16)
    scale = jnp.dot(drv, o_ref[...], preferred_element_type=F32)
    s = (p_ref[0] + p_ref[1]) * scale + b2_ref[...]
    out_ref[...] = s[:, :40]


def _t3(p2, drec2, O2, b2p):
    BT = 2000
    return pl.pallas_call(
        _t3_body,
        grid=(N // BT,),
        in_specs=[
            pl.BlockSpec((2, BT, 48), lambda i: (0, i, 0)),
            pl.BlockSpec((2, BT, 16), lambda i: (0, i, 0)),
            pl.BlockSpec((16, 48), lambda i: (0, 0)),
            pl.BlockSpec((1, 48), lambda i: (0, 0)),
        ],
        out_specs=pl.BlockSpec((BT, 40), lambda i: (i, 0)),
        out_shape=jax.ShapeDtypeStruct((N, 40), F32),
    )(p2, drec2, O2, b2p)


_MESH = plsc.VectorSubcoreMesh(
    core_axis_name="c", subcore_axis_name="s", num_cores=NC, num_subcores=NS)


def _sca_body(srcA_h, dstA_h, ts_h, td_h, e_h, dp_h,
              idxs_v, idxd_v, gs_v, gd_v, ev_v, zb_v, acc_sh,
              sg0, sg1, sd0, sd1):
    c = lax.axis_index("c")
    s = lax.axis_index("s")
    w = s * NC + c
    sgs = (sg0, sg1)
    sds = (sd0, sd1)

    def zrow(i, _):
        zb_v[i, :] = jnp.zeros((L,), F32)
        return 0
    lax.fori_loop(0, ROWS_PT, zrow, 0)
    pltpu.sync_copy(zb_v, acc_sh.at[pl.ds(s * ROWS_PT, ROWS_PT)])
    pltpu.sync_copy(srcA_h.at[w], idxs_v)
    pltpu.sync_copy(dstA_h.at[w], idxd_v)
    plsc.subcore_barrier()

    for q in range(2):
        pltpu.async_copy(ts_h.at[idxs_v.at[q]], gs_v.at[q], sgs[q])
        pltpu.async_copy(td_h.at[idxd_v.at[q]], gd_v.at[q], sds[q])

    def group(g, _):
        for q in range(2):
            b = 2 * g + q
            pltpu.make_async_copy(
                ts_h.at[idxs_v.at[0]], gs_v.at[q], sgs[q]).wait()
            pltpu.make_async_copy(
                td_h.at[idxd_v.at[0]], gd_v.at[q], sds[q]).wait()

            def erow(i, _):
                a = gs_v[q, i, :] + gd_v[q, i, :]
                a = jnp.where(a > 0, a, 0.2 * a)
                ev_v[i, :] = jnp.exp(a)
                return 0
            lax.fori_loop(0, BLK, erow, 0)
            pltpu.sync_copy(ev_v, e_h.at[pl.ds(w * EPT + b * BLK, BLK)])
            pltpu.sync_copy(ev_v, acc_sh.at[idxd_v.at[b]], add=True)
            nb = b + 2

            @pl.when(nb < NBLK)
            def _():
                pltpu.async_copy(ts_h.at[idxs_v.at[nb]], gs_v.at[q], sgs[q])
                pltpu.async_copy(td_h.at[idxd_v.at[nb]], gd_v.at[q], sds[q])
        return 0
    lax.fori_loop(0, NBLK // 2, group, 0)
    plsc.subcore_barrier()
    pltpu.sync_copy(acc_sh.at[pl.ds(s * ROWS_PT, ROWS_PT)],
                    dp_h.at[c, pl.ds(s * ROWS_PT, ROWS_PT)])


_sca = pl.kernel(
    _sca_body,
    out_type=[
        jax.ShapeDtypeStruct((EP, 16), F32),
        jax.ShapeDtypeStruct((2, NP, 16), F32),
    ],
    mesh=_MESH,
    compiler_params=_SC_PARAMS,
    scratch_types=[
        pltpu.VMEM((NBLK, BLK), jnp.int32),
        pltpu.VMEM((NBLK, BLK), jnp.int32),
        pltpu.VMEM((2, BLK, 16), F32),
        pltpu.VMEM((2, BLK, 16), F32),
        pltpu.VMEM((BLK, 16), F32),
        pltpu.VMEM((ROWS_PT, 16), F32),
        pltpu.VMEM_SHARED((NP, 16), F32),
        pltpu.SemaphoreType.DMA,
        pltpu.SemaphoreType.DMA,
        pltpu.SemaphoreType.DMA,
        pltpu.SemaphoreType.DMA,
    ],
)


def _scb_body(srcB_h, dstB_h, e_h, h4_h, out_h,
              idxs_v, idxd_v, ev_v, rows_v,
              si0, si1, si2, si3, sr0, sr1, sr2, se0, se1, ss0, ss1, ss2,
              acc_sh):
    c = lax.axis_index("c")
    s = lax.axis_index("s")
    sis = (si0, si1, si2, si3)
    srs = (sr0, sr1, sr2)
    ses = (se0, se1)
    sss = (ss0, ss1, ss2)

    for p in range(2):
        def zrow(i, _):
            for j in range(8):
                rows_v[0, i, pl.ds(16 * j, 16)] = jnp.zeros((L,), F32)
            return 0
        lax.fori_loop(0, BLKB, zrow, 0)
        for q in range(RPT_A // BLKB):
            pltpu.sync_copy(rows_v.at[0],
                            acc_sh.at[pl.ds(s * RPT_A + q * BLKB, BLKB)])
        rem = RPT_A % BLKB
        if rem:
            pltpu.sync_copy(
                rows_v.at[0, pl.ds(0, rem)],
                acc_sh.at[pl.ds(s * RPT_A + (RPT_A // BLKB) * BLKB, rem)])
        plsc.subcore_barrier()

        for cc in range(2):
            @pl.when(c == cc)
            def _(p=p, cc=cc):
                k = 2 * cc + p
                for j in range(2):
                    pltpu.sync_copy(srcB_h.at[s, j], idxs_v.at[j])
                    pltpu.sync_copy(dstB_h.at[s, j], idxd_v.at[j])
                pltpu.async_copy(srcB_h.at[s, 2], idxs_v.at[2], sis[2])
                pltpu.async_copy(dstB_h.at[s, 2], idxd_v.at[2], sis[2])
                for q in range(2):
                    pltpu.async_copy(h4_h.at[k].at[idxs_v.at[q]],
                                     rows_v.at[q], srs[q])
                    pltpu.async_copy(e_h.at[pl.ds(s * EPT2 + q * BLKB, BLKB)],
                                     ev_v.at[q], ses[q])

                def group(g, _):
                    for q in range(12):
                        b = 12 * g + q
                        r = q % 3
                        e2 = q % 2

                        @pl.when(b + 2 < NBB)
                        def _():
                            rn = (q + 2) % 3
                            i2 = (q + 2) % 4
                            pltpu.make_async_copy(
                                srcB_h.at[s, 0], idxs_v.at[i2], sis[i2]).wait()
                            pltpu.make_async_copy(
                                dstB_h.at[s, 0], idxd_v.at[i2], sis[i2]).wait()

                            @pl.when(b >= 1)
                            def _():
                                pltpu.make_async_copy(
                                    rows_v.at[rn], acc_sh.at[idxd_v.at[0]],
                                    sss[rn]).wait()
                            pltpu.async_copy(
                                h4_h.at[k].at[idxs_v.at[i2]], rows_v.at[rn],
                                srs[rn])

                        @pl.when(b + 3 < NBB)
                        def _():
                            i3 = (q + 3) % 4
                            pltpu.async_copy(srcB_h.at[s, b + 3],
                                             idxs_v.at[i3], sis[i3])
                            pltpu.async_copy(dstB_h.at[s, b + 3],
                                             idxd_v.at[i3], sis[i3])

                        pltpu.make_async_copy(
                            h4_h.at[k].at[idxs_v.at[0]], rows_v.at[r],
                            srs[r]).wait()
                        pltpu.make_async_copy(
                            e_h.at[pl.ds(0, BLKB)], ev_v.at[e2],
                            ses[e2]).wait()

                        def edge(i, _):
                            av = ev_v[e2, i, :]
                            b0 = jnp.broadcast_to(av[2 * k], (L,))
                            b1 = jnp.broadcast_to(av[2 * k + 1], (L,))
                            for j in range(4):
                                rows_v[r, i, pl.ds(16 * j, 16)] = (
                                    rows_v[r, i, pl.ds(16 * j, 16)] * b0)
                            for j in range(4, 8):
                                rows_v[r, i, pl.ds(16 * j, 16)] = (
                                    rows_v[r, i, pl.ds(16 * j, 16)] * b1)
                            return 0
                        lax.fori_loop(0, BLKB, edge, 0, unroll=2)
                        pltpu.async_copy(rows_v.at[r],
                                         acc_sh.at[idxd_v.at[q % 4]], sss[r],
                                         add=True)

                        @pl.when(b + 2 < NBB)
                        def _():
                            pltpu.async_copy(
                                e_h.at[pl.ds(s * EPT2 + (b + 2) * BLKB, BLKB)],
                                ev_v.at[(q + 2) % 2], ses[(q + 2) % 2])
                    return 0
                lax.fori_loop(0, NBB // 12, group, 0)
                for r in range(3):
                    pltpu.make_async_copy(
                        rows_v.at[r], acc_sh.at[idxd_v.at[0]], sss[r]).wait()
        plsc.subcore_barrier()
        k_dyn = 2 * c + p
        pltpu.sync_copy(acc_sh.at[pl.ds(s * RPT_A, RPT_A)],
                        out_h.at[k_dyn, pl.ds(s * RPT_A, RPT_A)])
        plsc.subcore_barrier()


_scb = pl.kernel(
    _scb_body,
    out_type=[jax.ShapeDtypeStruct((4, NP, 128), F32)],
    mesh=_MESH,
    compiler_params=_SC_PARAMS,
    scratch_types=[
        pltpu.VMEM((4, BLKB), jnp.int32),
        pltpu.VMEM((4, BLKB), jnp.int32),
        pltpu.VMEM((2, BLKB, 16), F32),
        pltpu.VMEM((3, BLKB, 128), F32),
        pltpu.SemaphoreType.DMA,
        pltpu.SemaphoreType.DMA,
        pltpu.SemaphoreType.DMA,
        pltpu.SemaphoreType.DMA,
        pltpu.SemaphoreType.DMA,
        pltpu.SemaphoreType.DMA,
        pltpu.SemaphoreType.DMA,
        pltpu.SemaphoreType.DMA,
        pltpu.SemaphoreType.DMA,
        pltpu.SemaphoreType.DMA,
        pltpu.SemaphoreType.DMA,
        pltpu.SemaphoreType.DMA,
        pltpu.VMEM_SHARED((NPA, 128), F32),
    ],
)


def _scc2_body(srcA_h, dstA_h, e_h, h2_h, p2_h,
               idxs_v, idxd_v, ev_v, rows_v,
               sr0, sr1, se0, se1, acc_sh):
    c = lax.axis_index("c")
    s = lax.axis_index("s")
    w = s * NC + c
    srs = (sr0, sr1)
    ses = (se0, se1)

    pltpu.sync_copy(srcA_h.at[w], idxs_v)
    pltpu.sync_copy(dstA_h.at[w], idxd_v)

    def zrow(i, _):
        for j in range(3):
            rows_v[0, i, pl.ds(16 * j, 16)] = jnp.zeros((L,), F32)
        return 0
    lax.fori_loop(0, BLK, zrow, 0)
    for q in range(5):
        pltpu.sync_copy(rows_v.at[0],
                        acc_sh.at[pl.ds(s * ROWS_PT + q * BLK, BLK)])
    plsc.subcore_barrier()

    for q in range(2):
        pltpu.async_copy(h2_h.at[idxs_v.at[q]], rows_v.at[q], srs[q])
        pltpu.async_copy(e_h.at[pl.ds(w * EPT + q * BLK, BLK)],
                         ev_v.at[q], ses[q])

    def group(g, _):
        for q in range(2):
            b = 2 * g + q
            pltpu.make_async_copy(
                h2_h.at[idxs_v.at[0]], rows_v.at[q], srs[q]).wait()
            pltpu.make_async_copy(
                e_h.at[pl.ds(0, BLK)], ev_v.at[q], ses[q]).wait()

            def edge(i, _):
                av = ev_v[q, i, :]
                b0 = jnp.broadcast_to(av[0], (L,))
                for j in range(3):
                    rows_v[q, i, pl.ds(16 * j, 16)] = (
                        rows_v[q, i, pl.ds(16 * j, 16)] * b0)
                return 0
            lax.fori_loop(0, BLK, edge, 0, unroll=2)
            pltpu.sync_copy(rows_v.at[q], acc_sh.at[idxd_v.at[b]], add=True)
            nb = b + 2

            @pl.when(nb < NBLK)
            def _():
                pltpu.async_copy(h2_h.at[idxs_v.at[nb]], rows_v.at[q], srs[q])
                pltpu.async_copy(e_h.at[pl.ds(w * EPT + nb * BLK, BLK)],
                                 ev_v.at[q], ses[q])
        return 0
    lax.fori_loop(0, NBLK // 2, group, 0)
    plsc.subcore_barrier()
    pltpu.sync_copy(acc_sh.at[pl.ds(s * ROWS_PT, ROWS_PT)],
                    p2_h.at[c, pl.ds(s * ROWS_PT, ROWS_PT)])


_scc2 = pl.kernel(
    _scc2_body,
    out_type=[jax.ShapeDtypeStruct((2, NP, 48), F32)],
    mesh=_MESH,
    compiler_params=_SC_PARAMS,
    scratch_types=[
        pltpu.VMEM((NBLK, BLK), jnp.int32),
        pltpu.VMEM((NBLK, BLK), jnp.int32),
        pltpu.VMEM((2, BLK, 16), F32),
        pltpu.VMEM((2, BLK, 48), F32),
        pltpu.SemaphoreType.DMA,
        pltpu.SemaphoreType.DMA,
        pltpu.SemaphoreType.DMA,
        pltpu.SemaphoreType.DMA,
        pltpu.VMEM_SHARED((NP, 48), F32),
    ],
)


@jax.jit
def kernel(x, edge_index, W1, a_src1, a_dst1, b1, W2, a_src2, a_dst2, b2):
    loop = jnp.arange(N, dtype=edge_index.dtype)
    padfill = PADN + jnp.arange(EP - E - N, dtype=edge_index.dtype) % (NP - N)
    srcp = jnp.concatenate([edge_index[0], loop, padfill])
    dstp = jnp.concatenate([edge_index[1], loop, padfill])
    srcA = srcp.reshape(NW, NBLK, BLK)
    dstA = dstp.reshape(NW, NBLK, BLK)
    srcB = srcp.reshape(NS, NBB, BLKB)
    dstB = dstp.reshape(NS, NBB, BLKB)
    xp = jnp.zeros((NP, F_IN), F32).at[:N].set(x)

    eye = jnp.eye(HEADS, dtype=F32)
    S_src = (a_src1[:, :, None] * eye[:, None, :]).reshape(512, 8)
    S_dst = (a_dst1[:, :, None] * eye[:, None, :]).reshape(512, 8)
    zc = jnp.zeros((512, 8), F32)
    Wsd = jnp.concatenate([S_src, zc, S_dst, zc], axis=1)

    b1r = jnp.pad(b1.reshape(4, 128), ((0, 4), (0, 0)))
    W2p = jnp.pad(W2, ((0, 0), (0, 8)))
    a2s = jnp.tile(jnp.pad(a_src2[0], (0, 8))[:, None], (1, 16))
    a2d = jnp.tile(jnp.pad(a_dst2[0], (0, 8))[:, None], (1, 16))
    A2 = jnp.concatenate([a2s, a2d], axis=1)
    b2p = jnp.pad(b2, (0, 8)).reshape(1, 48)

    R1m = jnp.pad(jnp.repeat(eye, 64, axis=1), ((0, 8), (0, 0)))
    O2m = jnp.zeros((16, 48), F32).at[0, :].set(1.0)

    h4, lg = _t1(xp, W1, Wsd)
    a1s, a1d = lg[:, :16], lg[:, 16:]
    e1, d1p = _sca(srcA, dstA, a1s, a1d)
    (out1,) = _scb(srcB, dstB, e1, h4)

    h2p, lg2 = _t2(out1, d1p, R1m, b1r, W2p, A2)
    a2s_t, a2d_t = lg2[:, :16], lg2[:, 16:]
    e2, d2p = _sca(srcA, dstA, a2s_t, a2d_t)
    (p2,) = _scc2(srcA, dstA, e2, h2p)

    return _t3(p2, d2p, O2m, b2p)
